# baseline (speedup 1.0000x reference)
import functools

import jax
import jax.numpy as jnp
import numpy as np
from jax import lax
from jax.experimental import pallas as pl
from jax.experimental.pallas import tpu as pltpu
from jax.experimental.pallas import tpu_sc as plsc

LATENT = 64
BATCH = 1024
TWOB = 2 * BATCH
REL = 1000
LMBDA = 0.1
_LOG2 = float(np.log(2.0))

_info = plsc.get_sparse_core_info()
_NC, _NS = _info.num_cores, _info.num_subcores
_NW = _NC * _NS
_BPW = TWOB // _NW


def _sc_gather_body(ent_hbm, bh_hbm, eh_out, idxh_v, rh_v, sem):
    wid = lax.axis_index("s") * _NC + lax.axis_index("c")
    base = wid * _BPW
    pltpu.sync_copy(bh_hbm.at[pl.ds(base, _BPW)], idxh_v)
    copies = []
    for g in range(_BPW // 16):
        vh = idxh_v[pl.ds(g * 16, 16)]
        for l in range(16):
            i = g * 16 + l
            copies.append(pltpu.async_copy(
                ent_hbm.at[pl.ds(vh[l], 1)], rh_v.at[pl.ds(i, 1)], sem))
    for c in copies:
        c.wait()
    pltpu.sync_copy(rh_v, eh_out.at[pl.ds(base, _BPW)])


_sc_gather = functools.partial(
    pl.kernel,
    out_type=jax.ShapeDtypeStruct((TWOB, LATENT), jnp.float32),
    mesh=plsc.VectorSubcoreMesh(core_axis_name="c", subcore_axis_name="s"),
    scratch_types=[
        pltpu.VMEM((_BPW,), jnp.int32),
        pltpu.VMEM((_BPW, LATENT), jnp.float32),
        pltpu.SemaphoreType.DMA,
    ],
)(_sc_gather_body)


def _tc_gather_body(bt_ref, ent_ref, et_ref, sem):
    def start(i, _):
        idx = bt_ref[i]
        pltpu.make_async_copy(
            ent_ref.at[pl.ds(idx, 1)], et_ref.at[pl.ds(i, 1)], sem).start()
        return 0

    lax.fori_loop(0, TWOB, start, 0)

    def drain(i, _):
        idx = bt_ref[i]
        pltpu.make_async_copy(
            ent_ref.at[pl.ds(idx, 1)], et_ref.at[pl.ds(i, 1)], sem).wait()
        return 0

    lax.fori_loop(0, TWOB, drain, 0)


def _finish_body(eh_ref, et_ref, rel_ref, br_ref, take2_ref,
                 loss_ref, s_ref):
    eh = eh_ref[...]
    et = et_ref[...]
    rel = rel_ref[...]
    br = br_ref[...]
    onehot = (br[:, None] ==
              lax.broadcasted_iota(jnp.int32, (TWOB, REL), 1)
              ).astype(jnp.float32)
    er = jnp.dot(onehot, rel, preferred_element_type=jnp.float32)
    s = jnp.sum(eh * et * er, axis=1)
    s_ref[...] = s
    a = jnp.abs(s)
    sp_pair = a + 2.0 * jnp.log1p(jnp.exp(-a))
    contrib = jnp.where(take2_ref[...] > 0, sp_pair, 2.0 * _LOG2)
    loss_main = jnp.sum(contrib) / (4.0 * BATCH)
    ssq = jnp.sum(eh * eh) + jnp.sum(et * et) + jnp.sum(er * er)
    regul = ssq / float(TWOB * LATENT)
    loss_ref[...] = jnp.broadcast_to(loss_main + LMBDA * regul, (1, 1))


def kernel(ent_embeddings, rel_embeddings, pos_h, pos_r, pos_t,
           neg_h, neg_r, neg_t, take):
    bh = jnp.concatenate([pos_h, neg_h]).astype(jnp.int32)
    bt = jnp.concatenate([pos_t, neg_t]).astype(jnp.int32)
    br = jnp.concatenate([pos_r, neg_r]).astype(jnp.int32)
    take2 = jnp.concatenate([take, take]).astype(jnp.float32)

    eh = _sc_gather(ent_embeddings, bh)

    et = pl.pallas_call(
        _tc_gather_body,
        in_specs=[
            pl.BlockSpec(memory_space=pltpu.MemorySpace.SMEM),
            pl.BlockSpec(memory_space=pltpu.MemorySpace.HBM),
        ],
        out_specs=pl.BlockSpec(memory_space=pltpu.MemorySpace.HBM),
        out_shape=jax.ShapeDtypeStruct((TWOB, LATENT), jnp.float32),
        scratch_shapes=[pltpu.SemaphoreType.DMA],
    )(bt, ent_embeddings)

    loss2d, s = pl.pallas_call(
        _finish_body,
        out_shape=[
            jax.ShapeDtypeStruct((1, 1), jnp.float32),
            jax.ShapeDtypeStruct((TWOB,), jnp.float32),
        ],
    )(eh, et, rel_embeddings, br, take2)
    return loss2d[0, 0], s[BATCH:]
